# Initial kernel scaffold; baseline (speedup 1.0000x reference)
import jax
import jax.numpy as jnp
from jax.experimental import pallas as pl


def kernel(x, Wg_s, Wu_s, Wd_s, Wg, Wu, Wd, Wr, rbias):
    raise NotImplementedError("write your pallas kernel here")


# speedup vs baseline: 1.2005x; 1.2005x over previous
import jax
import jax.numpy as jnp
from jax.experimental import pallas as pl
from jax.experimental.pallas import tpu as pltpu

S, H, I = 2048, 768, 384
E = 7
EP = 128
NEG = -1e30


def _dense_body(xr, wrr, rbr, wgsr, wusr, wdsr, wgr, wur, wdr, outr, wfull):
    e = pl.program_id(0)
    x = xr[...]

    @pl.when(e == 0)
    def _():
        h = jax.nn.silu(x @ wgsr[...]) * (x @ wusr[...])
        outr[...] = h @ wdsr[...]
        probs = jax.nn.sigmoid(x @ wrr[...] + rbr[...])
        lane = jax.lax.broadcasted_iota(jnp.int32, (S, EP), 1)
        m0 = jnp.max(probs, axis=1, keepdims=True)
        i0 = jnp.min(jnp.where(probs == m0, lane, EP), axis=1, keepdims=True)
        probs1 = jnp.where(lane == i0, NEG, probs)
        m1 = jnp.max(probs1, axis=1, keepdims=True)
        i1 = jnp.min(jnp.where(probs1 == m1, lane, EP), axis=1, keepdims=True)
        wfull[...] = m0 * (lane == i0) + m1 * (lane == i1)

    onehot = (jax.lax.broadcasted_iota(jnp.int32, (EP, 1), 0) == e).astype(jnp.float32)
    w_e = wfull[...] @ onehot
    h = jax.nn.silu(x @ wgr[0]) * (x @ wur[0])
    outr[...] += (h @ wdr[0]) * w_e


def kernel(x, Wg_s, Wu_s, Wd_s, Wg, Wu, Wd, Wr, rbias):
    xf = x.reshape(S, H)
    Wrp = jnp.zeros((H, EP), jnp.float32).at[:, :E].set(Wr)
    rbp = jnp.full((1, EP), NEG, jnp.float32).at[0, :E].set(rbias)

    out = pl.pallas_call(
        _dense_body,
        grid=(E,),
        in_specs=[
            pl.BlockSpec((S, H), lambda e: (0, 0)),
            pl.BlockSpec((H, EP), lambda e: (0, 0)),
            pl.BlockSpec((1, EP), lambda e: (0, 0)),
            pl.BlockSpec((H, I), lambda e: (0, 0)),
            pl.BlockSpec((H, I), lambda e: (0, 0)),
            pl.BlockSpec((I, H), lambda e: (0, 0)),
            pl.BlockSpec((1, H, I), lambda e: (e, 0, 0)),
            pl.BlockSpec((1, H, I), lambda e: (e, 0, 0)),
            pl.BlockSpec((1, I, H), lambda e: (e, 0, 0)),
        ],
        out_specs=pl.BlockSpec((S, H), lambda e: (0, 0)),
        out_shape=jax.ShapeDtypeStruct((S, H), jnp.float32),
        scratch_shapes=[pltpu.VMEM((S, EP), jnp.float32)],
        compiler_params=pltpu.CompilerParams(
            dimension_semantics=("arbitrary",),
        ),
    )(xf, Wrp, rbp, Wg_s, Wu_s, Wd_s, Wg, Wu, Wd)
    return out.reshape(1, S, H)
